# Initial kernel scaffold; baseline (speedup 1.0000x reference)
import jax
import jax.numpy as jnp
from jax.experimental import pallas as pl


def kernel(x, table):
    raise NotImplementedError("write your pallas kernel here")


# speedup vs baseline: 3.2758x; 3.2758x over previous
import functools

import jax
import jax.numpy as jnp
from jax import lax
from jax.experimental import pallas as pl
from jax.experimental.pallas import tpu as pltpu
from jax.experimental.pallas import tpu_sc as plsc

EMB = 128
NUM_ROWS = 16384 * 200
NUM_WORKERS = 32
ROWS_PER_WORKER = NUM_ROWS // NUM_WORKERS
CHUNK = 800
STEPS = ROWS_PER_WORKER // CHUNK


def _sc_body(idx_hbm, table_hbm, out_hbm, idx_v, rows_v, sem):
    wid = lax.axis_index("s") * 2 + lax.axis_index("c")
    base = wid * ROWS_PER_WORKER

    def step(i, carry):
        off = base + i * CHUNK
        pltpu.sync_copy(idx_hbm.at[pl.ds(off, CHUNK)], idx_v)
        pltpu.async_copy(table_hbm.at[idx_v], rows_v, sem).wait()
        pltpu.sync_copy(rows_v, out_hbm.at[pl.ds(off, CHUNK)])
        return carry

    lax.fori_loop(0, STEPS, step, 0)


_sc_gather = functools.partial(
    pl.kernel,
    mesh=plsc.VectorSubcoreMesh(core_axis_name="c", subcore_axis_name="s"),
    out_type=jax.ShapeDtypeStruct((NUM_ROWS, EMB), jnp.float32),
    scratch_types=[
        pltpu.VMEM((CHUNK,), jnp.int32),
        pltpu.VMEM((CHUNK, EMB), jnp.float32),
        pltpu.SemaphoreType.DMA,
    ],
)(_sc_body)


def kernel(x, table):
    flat = _sc_gather(x.reshape(-1), table)
    return flat.reshape(x.shape + (EMB,))


# speedup vs baseline: 11.6209x; 3.5475x over previous
import functools

import jax
import jax.numpy as jnp
from jax import lax
from jax.experimental import pallas as pl
from jax.experimental.pallas import tpu as pltpu
from jax.experimental.pallas import tpu_sc as plsc

EMB = 128
VOCAB = 84
NUM_ROWS = 16384 * 200
NUM_WORKERS = 32
ROWS_PER_WORKER = NUM_ROWS // NUM_WORKERS
CHUNK = 800
STEPS = ROWS_PER_WORKER // CHUNK


def _sc_body(idx_hbm, table_hbm, out_hbm, table_sp, idx_v, rows_v, sem):
    sid = lax.axis_index("s")
    wid = sid * 2 + lax.axis_index("c")
    base = wid * ROWS_PER_WORKER

    @pl.when(sid == 0)
    def _stage():
        pltpu.sync_copy(table_hbm, rows_v.at[pl.ds(0, VOCAB)])
        pltpu.sync_copy(rows_v.at[pl.ds(0, VOCAB)], table_sp)

    plsc.subcore_barrier()

    def step(i, carry):
        off = base + i * CHUNK
        pltpu.sync_copy(idx_hbm.at[pl.ds(off, CHUNK)], idx_v)
        pltpu.async_copy(table_sp.at[idx_v], rows_v, sem).wait()
        pltpu.sync_copy(rows_v, out_hbm.at[pl.ds(off, CHUNK)])
        return carry

    lax.fori_loop(0, STEPS, step, 0)


_sc_gather = functools.partial(
    pl.kernel,
    mesh=plsc.VectorSubcoreMesh(core_axis_name="c", subcore_axis_name="s"),
    out_type=jax.ShapeDtypeStruct((NUM_ROWS, EMB), jnp.float32),
    scratch_types=[
        pltpu.VMEM_SHARED((VOCAB, EMB), jnp.float32),
        pltpu.VMEM((CHUNK,), jnp.int32),
        pltpu.VMEM((CHUNK, EMB), jnp.float32),
        pltpu.SemaphoreType.DMA,
    ],
)(_sc_body)


def kernel(x, table):
    flat = _sc_gather(x.reshape(-1), table)
    return flat.reshape(x.shape + (EMB,))


# speedup vs baseline: 19.0102x; 1.6359x over previous
import functools

import jax
import jax.numpy as jnp
from jax import lax
from jax.experimental import pallas as pl
from jax.experimental.pallas import tpu as pltpu
from jax.experimental.pallas import tpu_sc as plsc

EMB = 128
VOCAB = 84
NUM_ROWS = 16384 * 200
NUM_WORKERS = 32
ROWS_PER_WORKER = NUM_ROWS // NUM_WORKERS
CHUNK = 400
STEPS = ROWS_PER_WORKER // CHUNK
J = STEPS // 2


def _sc_body(idx_hbm, table_hbm, out_hbm, table_sp,
             idx0, idx1, rows0, rows1,
             isem0, isem1, gsem0, gsem1, ssem0, ssem1):
    sid = lax.axis_index("s")
    wid = sid * 2 + lax.axis_index("c")
    base = wid * ROWS_PER_WORKER

    @pl.when(sid == 0)
    def _stage():
        pltpu.sync_copy(table_hbm, rows0.at[pl.ds(0, VOCAB)])
        pltpu.sync_copy(rows0.at[pl.ds(0, VOCAB)], table_sp)

    plsc.subcore_barrier()

    def idx_start(i, buf, sem):
        pltpu.async_copy(idx_hbm.at[pl.ds(base + i * CHUNK, CHUNK)], buf, sem)

    def store_start(i, buf, sem):
        pltpu.async_copy(buf, out_hbm.at[pl.ds(base + i * CHUNK, CHUNK)], sem)

    idx_start(0, idx0, isem0)
    idx_start(1, idx1, isem1)
    pltpu.make_async_copy(idx_hbm.at[pl.ds(0, CHUNK)], idx0, isem0).wait()
    pltpu.async_copy(table_sp.at[idx0], rows0, gsem0)

    def pair(j, carry):
        i0 = 2 * j
        not_last = j < J - 1

        pltpu.make_async_copy(table_sp.at[idx0], rows0, gsem0).wait()
        store_start(i0, rows0, ssem0)

        @pl.when(not_last)
        def _():
            idx_start(i0 + 2, idx0, isem0)

        pltpu.make_async_copy(idx_hbm.at[pl.ds(0, CHUNK)], idx1, isem1).wait()

        @pl.when(j >= 1)
        def _():
            pltpu.make_async_copy(rows1, out_hbm.at[pl.ds(0, CHUNK)], ssem1).wait()

        pltpu.async_copy(table_sp.at[idx1], rows1, gsem1)

        pltpu.make_async_copy(table_sp.at[idx1], rows1, gsem1).wait()
        store_start(i0 + 1, rows1, ssem1)

        @pl.when(not_last)
        def _():
            idx_start(i0 + 3, idx1, isem1)
            pltpu.make_async_copy(idx_hbm.at[pl.ds(0, CHUNK)], idx0, isem0).wait()
            pltpu.make_async_copy(rows0, out_hbm.at[pl.ds(0, CHUNK)], ssem0).wait()
            pltpu.async_copy(table_sp.at[idx0], rows0, gsem0)

        return carry

    lax.fori_loop(0, J, pair, 0)

    pltpu.make_async_copy(rows0, out_hbm.at[pl.ds(0, CHUNK)], ssem0).wait()
    pltpu.make_async_copy(rows1, out_hbm.at[pl.ds(0, CHUNK)], ssem1).wait()


_sc_gather = functools.partial(
    pl.kernel,
    mesh=plsc.VectorSubcoreMesh(core_axis_name="c", subcore_axis_name="s"),
    out_type=jax.ShapeDtypeStruct((NUM_ROWS, EMB), jnp.float32),
    scratch_types=[
        pltpu.VMEM_SHARED((VOCAB, EMB), jnp.float32),
        pltpu.VMEM((CHUNK,), jnp.int32),
        pltpu.VMEM((CHUNK,), jnp.int32),
        pltpu.VMEM((CHUNK, EMB), jnp.float32),
        pltpu.VMEM((CHUNK, EMB), jnp.float32),
        pltpu.SemaphoreType.DMA,
        pltpu.SemaphoreType.DMA,
        pltpu.SemaphoreType.DMA,
        pltpu.SemaphoreType.DMA,
        pltpu.SemaphoreType.DMA,
        pltpu.SemaphoreType.DMA,
    ],
)(_sc_body)


def kernel(x, table):
    flat = _sc_gather(x.reshape(-1), table)
    return flat.reshape(x.shape + (EMB,))
